# Initial kernel scaffold; baseline (speedup 1.0000x reference)
import jax
import jax.numpy as jnp
from jax.experimental import pallas as pl


def kernel(x, edge_index, batch, W0, b0, g0, be0, W1, b1, g1, be1, W2, b2, g2, be2, W3, b3, g3, be3, W4, b4, g4, be4, lw1, lb1, lw2, lb2):
    raise NotImplementedError("write your pallas kernel here")


# speedup vs baseline: 5.5390x; 5.5390x over previous
import functools

import jax
import jax.numpy as jnp
from jax import lax
from jax.experimental import pallas as pl
from jax.experimental.pallas import tpu as pltpu
from jax.experimental.pallas import tpu_sc as plsc

N = 10000
E = 320000
D = 128
G = 64
NL = 5

NW = 32
RPT = 80
EPAD = NW * RPT * 128
ACC_ROWS = 10112
ZPT = 632


def _mesh():
    return plsc.VectorSubcoreMesh(core_axis_name="c", subcore_axis_name="s")


def _sc_deg(dst2d, z16, ones16):
    @functools.partial(
        pl.kernel,
        mesh=_mesh(),
        out_type=jax.ShapeDtypeStruct((2, ACC_ROWS, 16), jnp.float32),
        scratch_types=[
            pltpu.VMEM((RPT, 128), jnp.int32),
            pltpu.VMEM((128, 16), jnp.float32),
            pltpu.VMEM_SHARED((ACC_ROWS, 16), jnp.float32),
        ],
    )
    def k(dst_hbm, z_hbm, ones_hbm, out_hbm, dst_v, ones_v, acc):
        c = lax.axis_index("c")
        s = lax.axis_index("s")
        wid = c * 16 + s
        pltpu.sync_copy(z_hbm, acc.at[pl.ds(s * ZPT, ZPT)])
        pltpu.sync_copy(ones_hbm, ones_v)
        pltpu.sync_copy(dst_hbm.at[pl.ds(wid * RPT, RPT)], dst_v)
        plsc.subcore_barrier()

        def body(j, carry):
            pltpu.sync_copy(ones_v, acc.at[dst_v.at[j]], add=True)
            return carry

        lax.fori_loop(0, RPT, body, 0)
        plsc.subcore_barrier()
        pltpu.sync_copy(acc.at[pl.ds(s * ZPT, ZPT)],
                        out_hbm.at[c, pl.ds(s * ZPT, ZPT)])

    return k(dst2d, z16, ones16)


def _sc_segsum(hp, src2d, dst2d, zrows):
    @functools.partial(
        pl.kernel,
        mesh=_mesh(),
        out_type=jax.ShapeDtypeStruct((2, ACC_ROWS, D), jnp.float32),
        scratch_types=[
            pltpu.VMEM((RPT, 128), jnp.int32),
            pltpu.VMEM((RPT, 128), jnp.int32),
            pltpu.VMEM((128, D), jnp.float32),
            pltpu.VMEM_SHARED((ACC_ROWS, D), jnp.float32),
        ],
    )
    def k(hp_hbm, src_hbm, dst_hbm, z_hbm, out_hbm, src_v, dst_v, rows_v, acc):
        c = lax.axis_index("c")
        s = lax.axis_index("s")
        wid = c * 16 + s
        pltpu.sync_copy(z_hbm, acc.at[pl.ds(s * ZPT, ZPT)])
        pltpu.sync_copy(src_hbm.at[pl.ds(wid * RPT, RPT)], src_v)
        pltpu.sync_copy(dst_hbm.at[pl.ds(wid * RPT, RPT)], dst_v)
        plsc.subcore_barrier()

        def body(j, carry):
            pltpu.sync_copy(hp_hbm.at[src_v.at[j]], rows_v)
            pltpu.sync_copy(rows_v, acc.at[dst_v.at[j]], add=True)
            return carry

        lax.fori_loop(0, RPT, body, 0)
        plsc.subcore_barrier()
        pltpu.sync_copy(acc.at[pl.ds(s * ZPT, ZPT)],
                        out_hbm.at[c, pl.ds(s * ZPT, ZPT)])

    return k(hp, src2d, dst2d, zrows)


def _tc_pre(x, degp, W0):
    def body(x_ref, degp_ref, w_ref, dinv_ref, hp_ref):
        deg = degp_ref[0, :N, 0:1] + degp_ref[1, :N, 0:1] + 1.0
        dinvb = jnp.broadcast_to(lax.rsqrt(deg), (N, D))
        dinv_ref[...] = dinvb
        h = jnp.dot(x_ref[...], w_ref[...], preferred_element_type=jnp.float32)
        hp_ref[...] = h * dinvb

    return pl.pallas_call(
        body,
        out_shape=(jax.ShapeDtypeStruct((N, D), jnp.float32),
                   jax.ShapeDtypeStruct((N, D), jnp.float32)),
    )(x, degp, W0)


def _tc_layer(i, x, P, hp, dinvb, b, g, be, Wn):
    has_res = i > 0
    has_next = Wn is not None

    def body(*refs):
        it = iter(refs)
        x_ref = next(it) if has_res else None
        p_ref, hp_ref, dinv_ref, b_ref, g_ref, be_ref = (next(it) for _ in range(6))
        w_ref = next(it) if has_next else None
        xo_ref = next(it)
        hpo_ref = next(it) if has_next else None

        dinvb_ = dinv_ref[...]
        t = (p_ref[0, :N] + p_ref[1, :N] + hp_ref[...]) * dinvb_ + b_ref[...]
        mu = jnp.mean(t, axis=0, keepdims=True)
        var = jnp.mean((t - mu) ** 2, axis=0, keepdims=True)
        t = (t - mu) * lax.rsqrt(var + 1e-5) * g_ref[...] + be_ref[...]
        t = jnp.maximum(t, 0.0)
        if has_res:
            t = x_ref[...] + t
        xo_ref[...] = t
        if has_next:
            hpo_ref[...] = jnp.dot(
                t, w_ref[...], preferred_element_type=jnp.float32) * dinvb_

    outs = [jax.ShapeDtypeStruct((N, D), jnp.float32)]
    if has_next:
        outs.append(jax.ShapeDtypeStruct((N, D), jnp.float32))
    args = []
    if has_res:
        args.append(x)
    args += [P, hp, dinvb, b.reshape(1, D), g.reshape(1, D), be.reshape(1, D)]
    if has_next:
        args.append(Wn)
    res = pl.pallas_call(body, out_shape=tuple(outs))(*args)
    return res if has_next else (res[0], None)


def _tc_pool(x, batch2d, lw1, lb1, lw2, lb2):
    def body(x_ref, b_ref, w1_ref, b1_ref, w2_ref, b2_ref, o_ref):
        gids = lax.broadcasted_iota(jnp.int32, (N, G), 1)
        onehot = (b_ref[...] == gids).astype(jnp.float32)
        sums = lax.dot_general(onehot, x_ref[...], (((0,), (0,)), ((), ())),
                               preferred_element_type=jnp.float32)
        cnt = jnp.sum(onehot, axis=0, keepdims=True)
        pooled = sums / jnp.clip(cnt, 1.0, None).T
        h = jnp.maximum(
            jnp.dot(pooled, w1_ref[...], preferred_element_type=jnp.float32)
            + b1_ref[...], 0.0)
        o_ref[...] = jnp.dot(
            h, w2_ref[...], preferred_element_type=jnp.float32) + b2_ref[...]

    return pl.pallas_call(
        body,
        out_shape=jax.ShapeDtypeStruct((G, 1), jnp.float32),
    )(x, batch2d, lw1, lb1.reshape(1, D // 2), lw2, lb2.reshape(1, 1))


def kernel(x, edge_index, batch, W0, b0, g0, be0, W1, b1, g1, be1, W2, b2, g2,
           be2, W3, b3, g3, be3, W4, b4, g4, be4, lw1, lb1, lw2, lb2):
    Ws = [W0, W1, W2, W3, W4]
    bs = [b0, b1, b2, b3, b4]
    gs = [g0, g1, g2, g3, g4]
    bes = [be0, be1, be2, be3, be4]

    pad = EPAD - E
    src2d = jnp.concatenate(
        [edge_index[0], jnp.zeros((pad,), jnp.int32)]).reshape(EPAD // 128, 128)
    dst2d = jnp.concatenate(
        [edge_index[1], jnp.full((pad,), N, jnp.int32)]).reshape(EPAD // 128, 128)
    z16 = jnp.zeros((ZPT, 16), jnp.float32)
    ones16 = jnp.ones((128, 16), jnp.float32)
    zrows = jnp.zeros((ZPT, D), jnp.float32)
    batch2d = batch.reshape(N, 1)

    degp = _sc_deg(dst2d, z16, ones16)
    dinvb, hp = _tc_pre(x, degp, W0)

    xcur = x
    for i in range(NL):
        P = _sc_segsum(hp, src2d, dst2d, zrows)
        Wn = Ws[i + 1] if i + 1 < NL else None
        xcur, hp = _tc_layer(i, xcur, P, hp, dinvb, bs[i], gs[i], bes[i], Wn)

    return _tc_pool(xcur, batch2d, lw1, lb1, lw2, lb2)
